# baseline (speedup 1.0000x reference)
import functools

import jax
import jax.numpy as jnp
from jax import lax
from jax.experimental import pallas as pl
from jax.experimental.pallas import tpu as pltpu
from jax.experimental.pallas import tpu_sc as plsc

N = 10000
N_PAD = 10240
D = 128
E = 320000
P = 4096
L = 16
NC, NS = 2, 16
NW = NC * NS
EPW = E // NW
ECH = 128
EFULL = EPW // ECH
EREM = EPW - EFULL * ECH
PPW = P // NW
RPW = N_PAD // NS
ZROW = N
PREC = lax.Precision.DEFAULT


def _tc_a_body(x_ref, wg_ref, ws_ref, pn_ref, plen_ref, ep_ref,
               xw_ref, xs_ref, pnf_ref, u_ref, v_ref, lf_ref):
    xb = x_ref[...]
    dg = functools.partial(lax.dot_general,
                           dimension_numbers=(((1,), (0,)), ((), ())),
                           precision=PREC)
    xw_ref[...] = dg(xb, wg_ref[...])
    xs_ref[...] = dg(xb, ws_ref[...])

    @pl.when(pl.program_id(0) == 0)
    def _():
        pos = lax.broadcasted_iota(jnp.int32, (P, L), 1)
        pid = lax.broadcasted_iota(jnp.int32, (P, L), 0)
        dummy = ZROW + ((pid * L + pos) & 127)
        plen = plen_ref[...]
        pnf_ref[...] = jnp.where(pos < plen, pn_ref[...], dummy)
        ep = ep_ref[...]
        u_ref[...] = ep[:, 0:1]
        v_ref[...] = ep[:, 1:2]
        lf_ref[...] = plen.astype(jnp.float32)


def _tc_a(x, wg, ws, path_nodes, plen2d, edge_pairs):
    blk = 1000
    return pl.pallas_call(
        _tc_a_body,
        grid=(N // blk,),
        in_specs=[
            pl.BlockSpec((blk, D), lambda i: (i, 0)),
            pl.BlockSpec((D, D), lambda i: (0, 0)),
            pl.BlockSpec((D, D), lambda i: (0, 0)),
            pl.BlockSpec((P, L), lambda i: (0, 0)),
            pl.BlockSpec((P, 1), lambda i: (0, 0)),
            pl.BlockSpec((P, 2), lambda i: (0, 0)),
        ],
        out_specs=[
            pl.BlockSpec((blk, D), lambda i: (i, 0)),
            pl.BlockSpec((blk, D), lambda i: (i, 0)),
            pl.BlockSpec((P, L), lambda i: (0, 0)),
            pl.BlockSpec((P, 1), lambda i: (0, 0)),
            pl.BlockSpec((P, 1), lambda i: (0, 0)),
            pl.BlockSpec((P, 1), lambda i: (0, 0)),
        ],
        out_shape=[
            jax.ShapeDtypeStruct((N, D), jnp.float32),
            jax.ShapeDtypeStruct((N, D), jnp.float32),
            jax.ShapeDtypeStruct((P, L), jnp.int32),
            jax.ShapeDtypeStruct((P, 1), jnp.int32),
            jax.ShapeDtypeStruct((P, 1), jnp.int32),
            jax.ShapeDtypeStruct((P, 1), jnp.float32),
        ],
    )(x, wg, ws, path_nodes, plen2d, edge_pairs)


def _sc_b_body(ei_hbm, xw_hbm,
               agg_hbm, deg0_hbm, deg1_hbm,
               sb0, sb1, db0, db1, ds0, ds1, srcr_v, dstr_v,
               rows0, rows1, ones_v, zbuf_v,
               acc_sh, deg_sh, gs0, gs1, is0, is1, ssem):
    c = lax.axis_index("c")
    s = lax.axis_index("s")
    zeros16 = jnp.zeros((16,), jnp.float32)
    ones16 = jnp.ones((16,), jnp.float32)
    rows = [rows0, rows1]
    srcb = [sb0, sb1]
    dstb = [db0, db1]
    dsts = [ds0, ds1]
    gsem = [gs0, gs1]
    isem = [is0, is1]

    @pl.loop(0, 32)
    def _(r):
        @pl.loop(0, D // 16)
        def _(ch):
            zbuf_v[r, pl.ds(ch * 16, 16)] = zeros16

    @pl.loop(0, ECH // 16)
    def _(j):
        ones_v[pl.ds(j * 16, 16)] = ones16

    @pl.loop(0, RPW // 32)
    def _(k):
        pltpu.sync_copy(zbuf_v, acc_sh.at[pl.ds(s * RPW + k * 32, 32)])

    @pl.loop(0, RPW // D)
    def _(k):
        pltpu.sync_copy(zbuf_v.at[0], deg_sh.at[pl.ds(s * RPW + k * D, D)])

    plsc.subcore_barrier()

    base = (c * NS + s) * EPW

    def idx_start(i, b):
        pltpu.async_copy(ei_hbm.at[pl.ds(base + i * ECH, ECH)], srcb[b],
                         isem[b])
        pltpu.async_copy(ei_hbm.at[pl.ds(E + base + i * ECH, ECH)], dstb[b],
                         isem[b])

    def idx_wait(i, b):
        pltpu.make_async_copy(ei_hbm.at[pl.ds(base + i * ECH, ECH)],
                              srcb[b], isem[b]).wait()
        pltpu.make_async_copy(ei_hbm.at[pl.ds(E + base + i * ECH, ECH)],
                              dstb[b], isem[b]).wait()

    def gather_start(b):
        pltpu.async_copy(xw_hbm.at[srcb[b]], rows[b], gsem[b])

    def gather_wait(b):
        pltpu.make_async_copy(xw_hbm.at[srcb[b]], rows[b], gsem[b]).wait()

    for b in range(2):
        idx_start(b, b)
    for b in range(2):
        idx_wait(b, b)
        gather_start(b)

    @pl.loop(0, EFULL - 2, step=2)
    def _(g):
        for b in range(2):
            i = g + b
            gather_wait(b)
            for j in range(ECH // 16):
                dsts[b][pl.ds(j * 16, 16)] = dstb[b][pl.ds(j * 16, 16)]
            sd = pltpu.async_copy(rows[b], acc_sh.at[dsts[b]], ssem, add=True)
            dd = pltpu.async_copy(ones_v, deg_sh.at[dsts[b]], ssem, add=True)
            idx_start(i + 2, b)
            sd.wait()
            dd.wait()
            idx_wait(i + 2, b)
            gather_start(b)

    for b in range(2):
        gather_wait(b)
        pltpu.sync_copy(rows[b], acc_sh.at[dstb[b]], add=True)
        pltpu.sync_copy(ones_v, deg_sh.at[dstb[b]], add=True)

    br = base + EFULL * ECH
    pltpu.sync_copy(ei_hbm.at[pl.ds(br, EREM)], srcr_v)
    pltpu.sync_copy(ei_hbm.at[pl.ds(E + br, EREM)], dstr_v)
    pltpu.async_copy(xw_hbm.at[srcr_v], rows0.at[pl.ds(0, EREM)], gs0).wait()
    pltpu.sync_copy(rows0.at[pl.ds(0, EREM)], acc_sh.at[dstr_v], add=True)
    pltpu.sync_copy(ones_v.at[pl.ds(0, EREM)], deg_sh.at[dstr_v], add=True)

    plsc.subcore_barrier()

    pltpu.sync_copy(acc_sh.at[pl.ds(s * RPW, RPW)],
                    agg_hbm.at[c, pl.ds(s * RPW, RPW)])

    @pl.when(c == 0)
    def _():
        pltpu.sync_copy(deg_sh.at[pl.ds(s * RPW, RPW)],
                        deg0_hbm.at[pl.ds(s * RPW, RPW)])

    @pl.when(c == 1)
    def _():
        pltpu.sync_copy(deg_sh.at[pl.ds(s * RPW, RPW)],
                        deg1_hbm.at[pl.ds(s * RPW, RPW)])


def _sc_b(ei_flat, xw):
    mesh = plsc.VectorSubcoreMesh(core_axis_name="c", subcore_axis_name="s")
    f = pl.kernel(
        _sc_b_body,
        out_type=[
            jax.ShapeDtypeStruct((NC, N_PAD, D), jnp.float32),
            jax.ShapeDtypeStruct((N_PAD,), jnp.float32),
            jax.ShapeDtypeStruct((N_PAD,), jnp.float32),
        ],
        mesh=mesh,
        scratch_types=[
            pltpu.VMEM((ECH,), jnp.int32),
            pltpu.VMEM((ECH,), jnp.int32),
            pltpu.VMEM((ECH,), jnp.int32),
            pltpu.VMEM((ECH,), jnp.int32),
            pltpu.VMEM((ECH,), jnp.int32),
            pltpu.VMEM((ECH,), jnp.int32),
            pltpu.VMEM((EREM,), jnp.int32),
            pltpu.VMEM((EREM,), jnp.int32),
            pltpu.VMEM((ECH, D), jnp.float32),
            pltpu.VMEM((ECH, D), jnp.float32),
            pltpu.VMEM((ECH,), jnp.float32),
            pltpu.VMEM((32, D), jnp.float32),
            pltpu.VMEM_SHARED((N_PAD, D), jnp.float32),
            pltpu.VMEM_SHARED((N_PAD,), jnp.float32),
            pltpu.SemaphoreType.DMA,
            pltpu.SemaphoreType.DMA,
            pltpu.SemaphoreType.DMA,
            pltpu.SemaphoreType.DMA,
            pltpu.SemaphoreType.DMA,
        ],
    )
    return f(ei_flat, xw)


def _tc_c_body(agg_ref, d0_ref, d1_ref, xs_ref, h_ref):
    i = pl.program_id(0)
    a = agg_ref[0] + agg_ref[1]
    deg = jnp.maximum(d0_ref[...] + d1_ref[...], 1.0)
    h = jax.nn.relu(a / deg + xs_ref[...])
    rid = lax.broadcasted_iota(jnp.int32, h.shape, 0) + i * h.shape[0]
    h_ref[...] = jnp.where(rid < N, h, 0.0)


def _tc_c(agg, deg0c, deg1c, xs):
    blk = 1024
    return pl.pallas_call(
        _tc_c_body,
        grid=(N_PAD // blk,),
        in_specs=[
            pl.BlockSpec((NC, blk, D), lambda i: (0, i, 0)),
            pl.BlockSpec((blk, 1), lambda i: (i, 0)),
            pl.BlockSpec((blk, 1), lambda i: (i, 0)),
            pl.BlockSpec((blk, D), lambda i: (i, 0)),
        ],
        out_specs=pl.BlockSpec((blk, D), lambda i: (i, 0)),
        out_shape=jax.ShapeDtypeStruct((N_PAD, D), jnp.float32),
    )(agg, deg0c, deg1c, xs)


def _sc_d_body(h_hbm, pnf_hbm, u_hbm, v_hbm,
               hu_hbm, hv_hbm, psum_hbm,
               idx_all, uvb, rb0, rb1, uvrows, out_v,
               gs0, gs1, usem):
    c = lax.axis_index("c")
    s = lax.axis_index("s")
    wid = c * NS + s
    pbase = wid * PPW
    rows = [rb0, rb1]
    gsem = [gs0, gs1]
    gp = ECH // L
    NG = PPW // gp

    pltpu.sync_copy(u_hbm.at[pl.ds(pbase, PPW)], uvb)
    ug = pltpu.async_copy(h_hbm.at[uvb], uvrows, usem)

    pltpu.sync_copy(pnf_hbm.at[pl.ds(pbase * L, PPW * L)], idx_all)

    def gather_start(g, b):
        pltpu.async_copy(h_hbm.at[idx_all.at[pl.ds(g * ECH, ECH)]],
                         rows[b], gsem[b])

    def gather_wait(g, b):
        pltpu.make_async_copy(h_hbm.at[idx_all.at[pl.ds(g * ECH, ECH)]],
                              rows[b], gsem[b]).wait()

    def sum_group(g, b):
        for p in range(gp):
            for ch in range(D // 16):
                sl = pl.ds(ch * 16, 16)
                v = [rows[b][p * L + j, sl] for j in range(L)]
                while len(v) > 1:
                    v = [v[k] + v[k + 1] for k in range(0, len(v) - 1, 2)] + \
                        ([v[-1]] if len(v) % 2 else [])
                out_v[g * gp + p, sl] = v[0]

    for b in range(2):
        gather_start(b, b)

    @pl.loop(0, NG - 2, step=2)
    def _(g):
        for b in range(2):
            gather_wait(g + b, b)
            sum_group(g + b, b)
            gather_start(g + b + 2, b)

    ug.wait()
    pltpu.sync_copy(uvrows, hu_hbm.at[pl.ds(pbase, PPW)])
    pltpu.sync_copy(v_hbm.at[pl.ds(pbase, PPW)], uvb)
    vg = pltpu.async_copy(h_hbm.at[uvb], uvrows, usem)

    for b in range(2):
        gather_wait(NG - 2 + b, b)
        sum_group(NG - 2 + b, b)

    pltpu.sync_copy(out_v, psum_hbm.at[pl.ds(pbase, PPW)])
    vg.wait()
    pltpu.sync_copy(uvrows, hv_hbm.at[pl.ds(pbase, PPW)])


def _sc_d(h_pad, pnf_flat, u_flat, v_flat):
    mesh = plsc.VectorSubcoreMesh(core_axis_name="c", subcore_axis_name="s")
    f = pl.kernel(
        _sc_d_body,
        out_type=[
            jax.ShapeDtypeStruct((P, D), jnp.float32),
            jax.ShapeDtypeStruct((P, D), jnp.float32),
            jax.ShapeDtypeStruct((P, D), jnp.float32),
        ],
        mesh=mesh,
        scratch_types=[
            pltpu.VMEM((PPW * L,), jnp.int32),
            pltpu.VMEM((PPW,), jnp.int32),
            pltpu.VMEM((ECH, D), jnp.float32),
            pltpu.VMEM((ECH, D), jnp.float32),
            pltpu.VMEM((PPW, D), jnp.float32),
            pltpu.VMEM((PPW, D), jnp.float32),
            pltpu.SemaphoreType.DMA,
            pltpu.SemaphoreType.DMA,
            pltpu.SemaphoreType.DMA,
        ],
    )
    return f(h_pad, pnf_flat, u_flat, v_flat)


def _tc_e_body(hu_ref, hv_ref, ps_ref, ln_ref,
               wp_ref, w1a_ref, w1b_ref, w1c_ref, w2_ref, out_ref):
    m = ps_ref[...] / ln_ref[...]
    dg = functools.partial(lax.dot_general,
                           dimension_numbers=(((1,), (0,)), ((), ())),
                           precision=PREC)
    hp = 2.0 * jax.nn.relu(dg(m, wp_ref[...]))
    z = jax.nn.relu(dg(hu_ref[...], w1a_ref[...])
                    + dg(hv_ref[...], w1b_ref[...])
                    + dg(hp, w1c_ref[...]))
    out_ref[...] = dg(z, w2_ref[...])


def _tc_e(hu, hv, psum, lens_f, wp, w_p1, w2):
    blk = 512
    return pl.pallas_call(
        _tc_e_body,
        grid=(P // blk,),
        in_specs=[
            pl.BlockSpec((blk, D), lambda i: (i, 0)),
            pl.BlockSpec((blk, D), lambda i: (i, 0)),
            pl.BlockSpec((blk, D), lambda i: (i, 0)),
            pl.BlockSpec((blk, 1), lambda i: (i, 0)),
            pl.BlockSpec((D, D), lambda i: (0, 0)),
            pl.BlockSpec((D, 256), lambda i: (0, 0)),
            pl.BlockSpec((D, 256), lambda i: (1, 0)),
            pl.BlockSpec((D, 256), lambda i: (2, 0)),
            pl.BlockSpec((256, 1), lambda i: (0, 0)),
        ],
        out_specs=pl.BlockSpec((blk, 1), lambda i: (i, 0)),
        out_shape=jax.ShapeDtypeStruct((P, 1), jnp.float32),
    )(hu, hv, psum, lens_f, wp, w_p1, w_p1, w_p1, w2)


def kernel(x, edge_index, edge_pairs, path_nodes, path_lengths,
           W_gnn, W_self, W_path, W_p1, W_p2):
    xw, xs, pnf, u2d, v2d, lens_f = _tc_a(
        x, W_gnn, W_self, path_nodes, path_lengths.reshape(P, 1), edge_pairs)
    agg, deg0, deg1 = _sc_b(edge_index.reshape(2 * E), xw)
    h_pad = _tc_c(agg, deg0.reshape(N_PAD, 1), deg1.reshape(N_PAD, 1), xs)
    hu, hv, psum = _sc_d(h_pad, pnf.reshape(-1), u2d.reshape(-1),
                         v2d.reshape(-1))
    return _tc_e(hu, hv, psum, lens_f, W_path, W_p1, W_p2)


# speedup vs baseline: 1.0791x; 1.0791x over previous
import functools

import jax
import jax.numpy as jnp
from jax import lax
from jax.experimental import pallas as pl
from jax.experimental.pallas import tpu as pltpu
from jax.experimental.pallas import tpu_sc as plsc

N = 10000
N_PAD = 10240
D = 128
E = 320000
P = 4096
L = 16
NC, NS = 2, 16
NW = NC * NS
EPW = E // NW
ECH = 128
EFULL = EPW // ECH
EREM = EPW - EFULL * ECH
PPW = P // NW
RPW = N_PAD // NS
ZROW = N
PREC = lax.Precision.DEFAULT


def _tc_a1_body(x_ref, wg_ref, xw_ref):
    xw_ref[...] = lax.dot_general(x_ref[...], wg_ref[...],
                                  (((1,), (0,)), ((), ())), precision=PREC)


def _tc_a1(x, wg):
    blk = 2000
    return pl.pallas_call(
        _tc_a1_body,
        grid=(N // blk,),
        in_specs=[
            pl.BlockSpec((blk, D), lambda i: (i, 0)),
            pl.BlockSpec((D, D), lambda i: (0, 0)),
        ],
        out_specs=pl.BlockSpec((blk, D), lambda i: (i, 0)),
        out_shape=jax.ShapeDtypeStruct((N, D), jnp.float32),
    )(x, wg)


def _tc_a2_body(x_ref, ws_ref, pn_ref, plen_ref, ep_ref,
                xs_ref, pnf_ref, u_ref, v_ref, lf_ref):
    xs_ref[...] = lax.dot_general(x_ref[...], ws_ref[...],
                                  (((1,), (0,)), ((), ())), precision=PREC)

    @pl.when(pl.program_id(0) == 0)
    def _():
        pos = lax.broadcasted_iota(jnp.int32, (P, L), 1)
        pid = lax.broadcasted_iota(jnp.int32, (P, L), 0)
        dummy = ZROW + ((pid * L + pos) & 127)
        plen = plen_ref[...]
        pnf_ref[...] = jnp.where(pos < plen, pn_ref[...], dummy)
        ep = ep_ref[...]
        u_ref[...] = ep[:, 0:1]
        v_ref[...] = ep[:, 1:2]
        lf_ref[...] = plen.astype(jnp.float32)


def _tc_a2(x, ws, path_nodes, plen2d, edge_pairs):
    blk = 2000
    return pl.pallas_call(
        _tc_a2_body,
        grid=(N // blk,),
        in_specs=[
            pl.BlockSpec((blk, D), lambda i: (i, 0)),
            pl.BlockSpec((D, D), lambda i: (0, 0)),
            pl.BlockSpec((P, L), lambda i: (0, 0)),
            pl.BlockSpec((P, 1), lambda i: (0, 0)),
            pl.BlockSpec((P, 2), lambda i: (0, 0)),
        ],
        out_specs=[
            pl.BlockSpec((blk, D), lambda i: (i, 0)),
            pl.BlockSpec((P, L), lambda i: (0, 0)),
            pl.BlockSpec((P, 1), lambda i: (0, 0)),
            pl.BlockSpec((P, 1), lambda i: (0, 0)),
            pl.BlockSpec((P, 1), lambda i: (0, 0)),
        ],
        out_shape=[
            jax.ShapeDtypeStruct((N, D), jnp.float32),
            jax.ShapeDtypeStruct((P, L), jnp.int32),
            jax.ShapeDtypeStruct((P, 1), jnp.int32),
            jax.ShapeDtypeStruct((P, 1), jnp.int32),
            jax.ShapeDtypeStruct((P, 1), jnp.float32),
        ],
    )(x, ws, path_nodes, plen2d, edge_pairs)


def _sc_b_body(ei_hbm, xw_hbm,
               agg_hbm, deg0_hbm, deg1_hbm,
               sb0, sb1, db0, db1, ds0, ds1, srcr_v, dstr_v,
               rows0, rows1, ones_v, zbuf_v,
               acc_sh, deg_sh, gs0, gs1, is0, is1, ssem):
    c = lax.axis_index("c")
    s = lax.axis_index("s")
    zeros16 = jnp.zeros((16,), jnp.float32)
    ones16 = jnp.ones((16,), jnp.float32)
    rows = [rows0, rows1]
    srcb = [sb0, sb1]
    dstb = [db0, db1]
    dsts = [ds0, ds1]
    gsem = [gs0, gs1]
    isem = [is0, is1]

    @pl.loop(0, 32)
    def _(r):
        @pl.loop(0, D // 16)
        def _(ch):
            zbuf_v[r, pl.ds(ch * 16, 16)] = zeros16

    @pl.loop(0, ECH // 16)
    def _(j):
        ones_v[pl.ds(j * 16, 16)] = ones16

    @pl.loop(0, RPW // 32)
    def _(k):
        pltpu.sync_copy(zbuf_v, acc_sh.at[pl.ds(s * RPW + k * 32, 32)])

    @pl.loop(0, RPW // D)
    def _(k):
        pltpu.sync_copy(zbuf_v.at[0], deg_sh.at[pl.ds(s * RPW + k * D, D)])

    base = (c * NS + s) * EPW

    def idx_start(i, b):
        pltpu.async_copy(ei_hbm.at[pl.ds(base + i * ECH, ECH)], srcb[b],
                         isem[b])
        pltpu.async_copy(ei_hbm.at[pl.ds(E + base + i * ECH, ECH)], dstb[b],
                         isem[b])

    def idx_wait(i, b):
        pltpu.make_async_copy(ei_hbm.at[pl.ds(base + i * ECH, ECH)],
                              srcb[b], isem[b]).wait()
        pltpu.make_async_copy(ei_hbm.at[pl.ds(E + base + i * ECH, ECH)],
                              dstb[b], isem[b]).wait()

    def gather_start(b):
        pltpu.async_copy(xw_hbm.at[srcb[b]], rows[b], gsem[b])

    def gather_wait(b):
        pltpu.make_async_copy(xw_hbm.at[srcb[b]], rows[b], gsem[b]).wait()

    for b in range(2):
        idx_start(b, b)
    for b in range(2):
        idx_wait(b, b)
        gather_start(b)

    plsc.subcore_barrier()

    @pl.loop(0, EFULL - 2, step=2)
    def _(g):
        for b in range(2):
            i = g + b
            gather_wait(b)
            for j in range(ECH // 16):
                dsts[b][pl.ds(j * 16, 16)] = dstb[b][pl.ds(j * 16, 16)]
            sd = pltpu.async_copy(rows[b], acc_sh.at[dsts[b]], ssem, add=True)
            dd = pltpu.async_copy(ones_v, deg_sh.at[dsts[b]], ssem, add=True)
            idx_start(i + 2, b)
            sd.wait()
            dd.wait()
            idx_wait(i + 2, b)
            gather_start(b)

    for b in range(2):
        gather_wait(b)
        pltpu.sync_copy(rows[b], acc_sh.at[dstb[b]], add=True)
        pltpu.sync_copy(ones_v, deg_sh.at[dstb[b]], add=True)

    br = base + EFULL * ECH
    pltpu.sync_copy(ei_hbm.at[pl.ds(br, EREM)], srcr_v)
    pltpu.sync_copy(ei_hbm.at[pl.ds(E + br, EREM)], dstr_v)
    pltpu.async_copy(xw_hbm.at[srcr_v], rows0.at[pl.ds(0, EREM)], gs0).wait()
    pltpu.sync_copy(rows0.at[pl.ds(0, EREM)], acc_sh.at[dstr_v], add=True)
    pltpu.sync_copy(ones_v.at[pl.ds(0, EREM)], deg_sh.at[dstr_v], add=True)

    plsc.subcore_barrier()

    pltpu.sync_copy(acc_sh.at[pl.ds(s * RPW, RPW)],
                    agg_hbm.at[c, pl.ds(s * RPW, RPW)])

    @pl.when(c == 0)
    def _():
        pltpu.sync_copy(deg_sh.at[pl.ds(s * RPW, RPW)],
                        deg0_hbm.at[pl.ds(s * RPW, RPW)])

    @pl.when(c == 1)
    def _():
        pltpu.sync_copy(deg_sh.at[pl.ds(s * RPW, RPW)],
                        deg1_hbm.at[pl.ds(s * RPW, RPW)])


def _sc_b(ei_flat, xw):
    mesh = plsc.VectorSubcoreMesh(core_axis_name="c", subcore_axis_name="s")
    f = pl.kernel(
        _sc_b_body,
        out_type=[
            jax.ShapeDtypeStruct((NC, N_PAD, D), jnp.float32),
            jax.ShapeDtypeStruct((N_PAD,), jnp.float32),
            jax.ShapeDtypeStruct((N_PAD,), jnp.float32),
        ],
        mesh=mesh,
        scratch_types=[
            pltpu.VMEM((ECH,), jnp.int32),
            pltpu.VMEM((ECH,), jnp.int32),
            pltpu.VMEM((ECH,), jnp.int32),
            pltpu.VMEM((ECH,), jnp.int32),
            pltpu.VMEM((ECH,), jnp.int32),
            pltpu.VMEM((ECH,), jnp.int32),
            pltpu.VMEM((EREM,), jnp.int32),
            pltpu.VMEM((EREM,), jnp.int32),
            pltpu.VMEM((ECH, D), jnp.float32),
            pltpu.VMEM((ECH, D), jnp.float32),
            pltpu.VMEM((ECH,), jnp.float32),
            pltpu.VMEM((32, D), jnp.float32),
            pltpu.VMEM_SHARED((N_PAD, D), jnp.float32),
            pltpu.VMEM_SHARED((N_PAD,), jnp.float32),
            pltpu.SemaphoreType.DMA,
            pltpu.SemaphoreType.DMA,
            pltpu.SemaphoreType.DMA,
            pltpu.SemaphoreType.DMA,
            pltpu.SemaphoreType.DMA,
        ],
    )
    return f(ei_flat, xw)


def _tc_c_body(agg_ref, d0_ref, d1_ref, xs_ref, h_ref):
    i = pl.program_id(0)
    a = agg_ref[0] + agg_ref[1]
    deg = jnp.maximum(d0_ref[...] + d1_ref[...], 1.0)
    h = jax.nn.relu(a / deg + xs_ref[...])
    rid = lax.broadcasted_iota(jnp.int32, h.shape, 0) + i * h.shape[0]
    h_ref[...] = jnp.where(rid < N, h, 0.0)


def _tc_c(agg, deg0c, deg1c, xs):
    blk = 2048
    return pl.pallas_call(
        _tc_c_body,
        grid=(N_PAD // blk,),
        in_specs=[
            pl.BlockSpec((NC, blk, D), lambda i: (0, i, 0)),
            pl.BlockSpec((blk, 1), lambda i: (i, 0)),
            pl.BlockSpec((blk, 1), lambda i: (i, 0)),
            pl.BlockSpec((blk, D), lambda i: (i, 0)),
        ],
        out_specs=pl.BlockSpec((blk, D), lambda i: (i, 0)),
        out_shape=jax.ShapeDtypeStruct((N_PAD, D), jnp.float32),
    )(agg, deg0c, deg1c, xs)


def _sc_d_body(h_hbm, pnf_hbm, u_hbm, v_hbm,
               hu_hbm, hv_hbm, psum_hbm,
               idx_all, uvb, rb0, rb1, uvrows, out_v,
               gs0, gs1, usem):
    c = lax.axis_index("c")
    s = lax.axis_index("s")
    wid = c * NS + s
    pbase = wid * PPW
    rows = [rb0, rb1]
    gsem = [gs0, gs1]
    gp = ECH // L
    NG = PPW // gp

    pltpu.sync_copy(u_hbm.at[pl.ds(pbase, PPW)], uvb)
    ug = pltpu.async_copy(h_hbm.at[uvb], uvrows, usem)

    pltpu.sync_copy(pnf_hbm.at[pl.ds(pbase * L, PPW * L)], idx_all)

    def gather_start(g, b):
        pltpu.async_copy(h_hbm.at[idx_all.at[pl.ds(g * ECH, ECH)]],
                         rows[b], gsem[b])

    def gather_wait(g, b):
        pltpu.make_async_copy(h_hbm.at[idx_all.at[pl.ds(g * ECH, ECH)]],
                              rows[b], gsem[b]).wait()

    def sum_group(g, b):
        for p in range(gp):
            for ch in range(D // 16):
                sl = pl.ds(ch * 16, 16)
                v = [rows[b][p * L + j, sl] for j in range(L)]
                while len(v) > 1:
                    v = [v[k] + v[k + 1] for k in range(0, len(v) - 1, 2)] + \
                        ([v[-1]] if len(v) % 2 else [])
                out_v[g * gp + p, sl] = v[0]

    for b in range(2):
        gather_start(b, b)

    @pl.loop(0, NG - 2, step=2)
    def _(g):
        for b in range(2):
            gather_wait(g + b, b)
            sum_group(g + b, b)
            gather_start(g + b + 2, b)

    ug.wait()
    pltpu.sync_copy(uvrows, hu_hbm.at[pl.ds(pbase, PPW)])
    pltpu.sync_copy(v_hbm.at[pl.ds(pbase, PPW)], uvb)
    vg = pltpu.async_copy(h_hbm.at[uvb], uvrows, usem)

    for b in range(2):
        gather_wait(NG - 2 + b, b)
        sum_group(NG - 2 + b, b)

    pltpu.sync_copy(out_v, psum_hbm.at[pl.ds(pbase, PPW)])
    vg.wait()
    pltpu.sync_copy(uvrows, hv_hbm.at[pl.ds(pbase, PPW)])


def _sc_d(h_pad, pnf_flat, u_flat, v_flat):
    mesh = plsc.VectorSubcoreMesh(core_axis_name="c", subcore_axis_name="s")
    f = pl.kernel(
        _sc_d_body,
        out_type=[
            jax.ShapeDtypeStruct((P, D), jnp.float32),
            jax.ShapeDtypeStruct((P, D), jnp.float32),
            jax.ShapeDtypeStruct((P, D), jnp.float32),
        ],
        mesh=mesh,
        scratch_types=[
            pltpu.VMEM((PPW * L,), jnp.int32),
            pltpu.VMEM((PPW,), jnp.int32),
            pltpu.VMEM((ECH, D), jnp.float32),
            pltpu.VMEM((ECH, D), jnp.float32),
            pltpu.VMEM((PPW, D), jnp.float32),
            pltpu.VMEM((PPW, D), jnp.float32),
            pltpu.SemaphoreType.DMA,
            pltpu.SemaphoreType.DMA,
            pltpu.SemaphoreType.DMA,
        ],
    )
    return f(h_pad, pnf_flat, u_flat, v_flat)


def _tc_e_body(hu_ref, hv_ref, ps_ref, ln_ref,
               wp_ref, w1a_ref, w1b_ref, w1c_ref, w2_ref, out_ref):
    m = ps_ref[...] / ln_ref[...]
    dg = functools.partial(lax.dot_general,
                           dimension_numbers=(((1,), (0,)), ((), ())),
                           precision=PREC)
    hp = 2.0 * jax.nn.relu(dg(m, wp_ref[...]))
    z = jax.nn.relu(dg(hu_ref[...], w1a_ref[...])
                    + dg(hv_ref[...], w1b_ref[...])
                    + dg(hp, w1c_ref[...]))
    out_ref[...] = dg(z, w2_ref[...])


def _tc_e(hu, hv, psum, lens_f, wp, w_p1, w2):
    blk = 512
    return pl.pallas_call(
        _tc_e_body,
        grid=(P // blk,),
        in_specs=[
            pl.BlockSpec((blk, D), lambda i: (i, 0)),
            pl.BlockSpec((blk, D), lambda i: (i, 0)),
            pl.BlockSpec((blk, D), lambda i: (i, 0)),
            pl.BlockSpec((blk, 1), lambda i: (i, 0)),
            pl.BlockSpec((D, D), lambda i: (0, 0)),
            pl.BlockSpec((D, 256), lambda i: (0, 0)),
            pl.BlockSpec((D, 256), lambda i: (1, 0)),
            pl.BlockSpec((D, 256), lambda i: (2, 0)),
            pl.BlockSpec((256, 1), lambda i: (0, 0)),
        ],
        out_specs=pl.BlockSpec((blk, 1), lambda i: (i, 0)),
        out_shape=jax.ShapeDtypeStruct((P, 1), jnp.float32),
    )(hu, hv, psum, lens_f, wp, w_p1, w_p1, w_p1, w2)


def kernel(x, edge_index, edge_pairs, path_nodes, path_lengths,
           W_gnn, W_self, W_path, W_p1, W_p2):
    xw = _tc_a1(x, W_gnn)
    agg, deg0, deg1 = _sc_b(edge_index.reshape(2 * E), xw)
    xs, pnf, u2d, v2d, lens_f = _tc_a2(
        x, W_self, path_nodes, path_lengths.reshape(P, 1), edge_pairs)
    h_pad = _tc_c(agg, deg0.reshape(N_PAD, 1), deg1.reshape(N_PAD, 1), xs)
    hu, hv, psum = _sc_d(h_pad, pnf.reshape(-1), u2d.reshape(-1),
                         v2d.reshape(-1))
    return _tc_e(hu, hv, psum, lens_f, W_path, W_p1, W_p2)
